# baseline (speedup 1.0000x reference)
import functools

import jax
import jax.numpy as jnp
from jax import lax
from jax.experimental import pallas as pl
from jax.experimental.pallas import tpu as pltpu
from jax.experimental.pallas import tpu_sc as plsc

_F = 26
_V = 20
_E = 64
_A = 20.0
_NV = _F * _V
_NVP = 528


def _tc_prep(xt_ref, primt_ref, dfltt_ref, cnt_ref, blendt_ref, fidxt_ref):
    c = cnt_ref[...].astype(jnp.float32)
    w = (c / (c + _A))[None, :]
    blendt_ref[...] = w * primt_ref[...] + (1.0 - w) * dfltt_ref[...]
    fofs = lax.broadcasted_iota(jnp.int32, xt_ref.shape, 0) * _V
    fidxt_ref[...] = xt_ref[...] + fofs


def kernel(X, emb_table, counts):
    B = X.shape[0]
    NBT = B // 128

    emb3 = emb_table.reshape(_F, _V + 1, _E)
    primt = jnp.transpose(emb3[:, 1:, :], (2, 0, 1)).reshape(_E, _NV)
    dfltt = jnp.repeat(jnp.transpose(emb3[:, 0, :], (1, 0)), _V, axis=1)
    primt = jnp.pad(primt, ((0, 0), (0, _NVP - _NV)))
    dfltt = jnp.pad(dfltt, ((0, 0), (0, _NVP - _NV)))
    cntp = jnp.pad(counts.reshape(_NV), (0, _NVP - _NV))
    XT = jnp.transpose(X, (1, 0))

    blendt, fidxt = pl.pallas_call(
        _tc_prep,
        out_shape=(
            jax.ShapeDtypeStruct((_E, _NVP), jnp.float32),
            jax.ShapeDtypeStruct((_F, B), jnp.int32),
        ),
    )(XT, primt, dfltt, cntp)

    info = plsc.get_sparse_core_info()
    NC, NS = info.num_cores, info.num_subcores
    NW = NC * NS
    NCHUNK = _F * NBT
    CPW = NCHUNK // NW
    fidx2 = fidxt.reshape(NCHUNK, 128)

    mesh = plsc.VectorSubcoreMesh(core_axis_name="c", subcore_axis_name="s")

    @functools.partial(
        pl.kernel,
        out_type=jax.ShapeDtypeStruct((_F, 8, NBT, 8, 128), jnp.float32),
        mesh=mesh,
        compiler_params=pltpu.CompilerParams(
            use_tc_tiling_on_sc=False, needs_layout_passes=False
        ),
        scratch_types=[
            pltpu.VMEM((_E, _NVP), jnp.float32),
            pltpu.VMEM((CPW, 128), jnp.int32),
            pltpu.VMEM((8, 8, 128), jnp.float32),
            pltpu.VMEM((8, 8, 128), jnp.float32),
            pltpu.SemaphoreType.DMA,
            pltpu.SemaphoreType.DMA,
        ],
    )
    def sc_fill(fidx_hbm, blendt_hbm, out_hbm, tbl_v, idx_v, obuf0, obuf1, sem0, sem1):
        wid = lax.axis_index("s") * NC + lax.axis_index("c")
        pltpu.sync_copy(blendt_hbm, tbl_v)
        pltpu.sync_copy(fidx_hbm.at[pl.ds(wid * CPW, CPW)], idx_v)

        def chunk(j, obuf, sem):
            t = wid * CPW + j
            f = t // NBT
            bt = t % NBT

            @pl.when(j >= 2)
            def _():
                tp = t - 2
                pltpu.make_async_copy(
                    obuf, out_hbm.at[tp // NBT, :, tp % NBT], sem
                ).wait()

            for c in range(8):
                idx16 = idx_v[j, pl.ds(c * 16, 16)]
                for e in range(_E):
                    vals = plsc.load_gather(
                        tbl_v, [jnp.full((16,), e, jnp.int32), idx16]
                    )
                    obuf[e // 8, e % 8, pl.ds(c * 16, 16)] = vals
            pltpu.async_copy(obuf, out_hbm.at[f, :, bt], sem)

        def body(i, carry):
            chunk(2 * i, obuf0, sem0)
            chunk(2 * i + 1, obuf1, sem1)
            return carry

        lax.fori_loop(0, CPW // 2, body, 0)
        tl0 = wid * CPW + CPW - 2
        tl1 = wid * CPW + CPW - 1
        pltpu.make_async_copy(obuf0, out_hbm.at[tl0 // NBT, :, tl0 % NBT], sem0).wait()
        pltpu.make_async_copy(obuf1, out_hbm.at[tl1 // NBT, :, tl1 % NBT], sem1).wait()

    q = sc_fill(fidx2, blendt)
    return q.transpose((2, 4, 0, 1, 3)).reshape(B, _F, _E)


# speedup vs baseline: 1.4754x; 1.4754x over previous
import functools

import jax
import jax.numpy as jnp
from jax import lax
from jax.experimental import pallas as pl
from jax.experimental.pallas import tpu as pltpu
from jax.experimental.pallas import tpu_sc as plsc

_F = 26
_V = 20
_E = 64
_A = 20.0
_NV = _F * _V
_NVP = 528


def _tc_prep(xt_ref, primt_ref, dfltt_ref, cnt_ref, blendt_ref, fidxt_ref):
    c = cnt_ref[...].astype(jnp.float32)
    w = (c / (c + _A))[None, :]
    blendt_ref[...] = w * primt_ref[...] + (1.0 - w) * dfltt_ref[...]
    fofs = lax.broadcasted_iota(jnp.int32, xt_ref.shape, 0) * _V
    fidxt_ref[...] = xt_ref[...] + fofs


def kernel(X, emb_table, counts):
    B = X.shape[0]
    NBT = B // 128

    emb3 = emb_table.reshape(_F, _V + 1, _E)
    primt = jnp.transpose(emb3[:, 1:, :], (2, 0, 1)).reshape(_E, _NV)
    dfltt = jnp.repeat(jnp.transpose(emb3[:, 0, :], (1, 0)), _V, axis=1)
    primt = jnp.pad(primt, ((0, 0), (0, _NVP - _NV)))
    dfltt = jnp.pad(dfltt, ((0, 0), (0, _NVP - _NV)))
    cntp = jnp.pad(counts.reshape(_NV), (0, _NVP - _NV))
    XT = jnp.transpose(X, (1, 0))

    blendt, fidxt = pl.pallas_call(
        _tc_prep,
        out_shape=(
            jax.ShapeDtypeStruct((_E, _NVP), jnp.float32),
            jax.ShapeDtypeStruct((_F, B), jnp.int32),
        ),
    )(XT, primt, dfltt, cntp)

    info = plsc.get_sparse_core_info()
    NC, NS = info.num_cores, info.num_subcores
    NW = NC * NS
    NCHUNK = _F * NBT
    CPW = NCHUNK // NW
    fidx2 = fidxt.reshape(NCHUNK, 128)

    mesh = plsc.VectorSubcoreMesh(core_axis_name="c", subcore_axis_name="s")

    @functools.partial(
        pl.kernel,
        out_type=jax.ShapeDtypeStruct((_F, 8, NBT, 8, 128), jnp.float32),
        mesh=mesh,
        compiler_params=pltpu.CompilerParams(
            use_tc_tiling_on_sc=False, needs_layout_passes=False
        ),
        scratch_types=[
            pltpu.VMEM((_E, _NVP), jnp.float32),
            pltpu.VMEM((CPW, 128), jnp.int32),
            pltpu.VMEM((8, 8, 128), jnp.float32),
            pltpu.VMEM((8, 8, 128), jnp.float32),
            pltpu.SemaphoreType.DMA,
            pltpu.SemaphoreType.DMA,
        ],
    )
    def sc_fill(fidx_hbm, blendt_hbm, out_hbm, tbl_v, idx_v, obuf0, obuf1, sem0, sem1):
        wid = lax.axis_index("s") * NC + lax.axis_index("c")
        pltpu.sync_copy(blendt_hbm, tbl_v)
        pltpu.sync_copy(fidx_hbm.at[pl.ds(wid * CPW, CPW)], idx_v)

        def chunk(j, obuf, sem):
            t = wid * CPW + j
            f = t // NBT
            bt = t % NBT

            @pl.when(j >= 2)
            def _():
                tp = t - 2
                pltpu.make_async_copy(
                    obuf, out_hbm.at[tp // NBT, :, tp % NBT], sem
                ).wait()

            for c in range(8):
                idx16 = idx_v[j, pl.ds(c * 16, 16)]
                for eg in range(0, _E, 8):
                    vals = [
                        plsc.load_gather(
                            tbl_v, [jnp.full((16,), e, jnp.int32), idx16]
                        )
                        for e in range(eg, eg + 8)
                    ]
                    for k, v in enumerate(vals):
                        e = eg + k
                        obuf[e // 8, e % 8, pl.ds(c * 16, 16)] = v
            pltpu.async_copy(obuf, out_hbm.at[f, :, bt], sem)

        def body(i, carry):
            chunk(2 * i, obuf0, sem0)
            chunk(2 * i + 1, obuf1, sem1)
            return carry

        lax.fori_loop(0, CPW // 2, body, 0)
        tl0 = wid * CPW + CPW - 2
        tl1 = wid * CPW + CPW - 1
        pltpu.make_async_copy(obuf0, out_hbm.at[tl0 // NBT, :, tl0 % NBT], sem0).wait()
        pltpu.make_async_copy(obuf1, out_hbm.at[tl1 // NBT, :, tl1 % NBT], sem1).wait()

    q = sc_fill(fidx2, blendt)
    return q.transpose((2, 4, 0, 1, 3)).reshape(B, _F, _E)
